# baseline (speedup 1.0000x reference)
import jax
import jax.numpy as jnp
from jax.experimental import pallas as pl

NUM_EXPERTS = 64
TOP_K = 8
BLOCK_R = 512


def _gate_kernel(x_ref, w_ref, ow_ref, oi_ref):
    logits = jax.lax.dot_general(
        w_ref[...], x_ref[...], (((1,), (1,)), ((), ())),
        preferred_element_type=jnp.float32,
    )
    iota = jax.lax.broadcasted_iota(jnp.int32, logits.shape, 0)
    rev = (NUM_EXPERTS - 1) - iota
    l = logits
    vals = []
    idxs = []
    for _ in range(TOP_K):
        m = jnp.max(l, axis=0, keepdims=True)
        enc = jnp.where(l == m, rev, 0)
        idx = (NUM_EXPERTS - 1) - jnp.max(enc, axis=0, keepdims=True)
        vals.append(m)
        idxs.append(idx)
        l = jnp.where(iota == idx, -jnp.inf, l)
    v = jnp.concatenate(vals, axis=0)
    i = jnp.concatenate(idxs, axis=0)
    e = jnp.exp(v - v[0:1, :])
    w8 = e / jnp.sum(e, axis=0, keepdims=True)
    ow_ref[...] = w8.T
    oi_ref[...] = i.T


def kernel(x, gate_weight):
    n, d = x.shape
    ow, oi = pl.pallas_call(
        _gate_kernel,
        grid=(n // BLOCK_R,),
        in_specs=[
            pl.BlockSpec((BLOCK_R, d), lambda i: (i, 0)),
            pl.BlockSpec((NUM_EXPERTS, d), lambda i: (0, 0)),
        ],
        out_specs=[
            pl.BlockSpec((BLOCK_R, TOP_K), lambda i: (i, 0)),
            pl.BlockSpec((BLOCK_R, TOP_K), lambda i: (i, 0)),
        ],
        out_shape=[
            jax.ShapeDtypeStruct((n, TOP_K), jnp.float32),
            jax.ShapeDtypeStruct((n, TOP_K), jnp.int32),
        ],
    )(x, gate_weight)
    return ow, oi
